# baseline (speedup 1.0000x reference)
import functools

import jax
import jax.numpy as jnp
from jax import lax
from jax.experimental import pallas as pl
from jax.experimental.pallas import tpu as pltpu
from jax.experimental.pallas import tpu_sc as plsc

N_NODES = 10000
DEG = 32
FEAT = 128
NHEADS = 4
OUT = 32
DAUG = 144
EDGES = DEG + 1

NC = 2
NS = 16
NW = NC * NS
GRP = 3
NT = 318
NPAD = NW * NT
NCHUNK = NT // GRP
IDXS = 104


def _mm_body(x_ref, w_ref, a_ref, o_ref):
    t = jnp.dot(x_ref[...], w_ref[...], preferred_element_type=jnp.float32)
    s = jnp.dot(t, a_ref[...], preferred_element_type=jnp.float32)
    o_ref[:, :FEAT] = t
    o_ref[:, FEAT:DAUG] = s


def _taug_matmul(x, w_all_t, afull):
    m, f = x.shape
    bm = 1000
    return pl.pallas_call(
        _mm_body,
        grid=(m // bm,),
        in_specs=[
            pl.BlockSpec((bm, f), lambda i: (i, 0)),
            pl.BlockSpec((f, FEAT), lambda i: (0, 0)),
            pl.BlockSpec((FEAT, DAUG - FEAT), lambda i: (0, 0)),
        ],
        out_specs=pl.BlockSpec((bm, DAUG), lambda i: (i, 0)),
        out_shape=jax.ShapeDtypeStruct((m, DAUG), jnp.float32),
    )(x, w_all_t, afull)


_LOG2E = 1.4426950408889634
_LN2_HI = 0.6931471824645996
_LN2_LO = -1.904654323148236e-09


def _exp_hi(v):
    t = v * _LOG2E
    tf = t + 0.5
    n = tf.astype(jnp.int32)
    nf = n.astype(jnp.float32)
    n = jnp.where(nf > tf, n - 1, n)
    nf = n.astype(jnp.float32)
    r = (v - nf * _LN2_HI) - nf * _LN2_LO
    p = jnp.float32(1.0 / 5040)
    for c in (1.0 / 720, 1.0 / 120, 1.0 / 24, 1.0 / 6, 0.5, 1.0, 1.0):
        p = p * r + c
    a = jnp.minimum(jnp.maximum(n + 30, 0), 30)
    b = jnp.minimum(jnp.maximum(n + 30 - a, 0), 30)
    one = jnp.full((16,), 1, jnp.int32)
    scale = (one << a).astype(jnp.float32) * (one << b).astype(jnp.float32)
    return (p * jnp.float32(2.0 ** -30)) * scale


def _compute_chunk(rows_v, out_v, lane):
    for i in range(GRP):
        r0 = i * EDGES
        srow_self = rows_v[r0, pl.ds(128, 16)]
        ssrc_vec = jnp.zeros((16,), jnp.float32)
        for h in range(NHEADS):
            ssrc_vec = jnp.where(lane == h, srow_self[4 + h], ssrc_vec)
        evs = []
        m = None
        for k in range(EDGES):
            srow = rows_v[r0 + k, pl.ds(128, 16)]
            sc = ssrc_vec + srow
            v = jnp.maximum(sc, 0.2 * sc)
            e = _exp_hi(v)
            evs.append(e)
            m = e if m is None else jnp.maximum(m, e)
        ps = [jnp.exp(evs[k] - m) for k in range(EDGES)]
        z = ps[0]
        for k in range(1, EDGES):
            z = z + ps[k]
        acc = [jnp.zeros((16,), jnp.float32) for _ in range(8)]
        for k in range(EDGES):
            p = ps[k]
            row = r0 + k
            for h in range(NHEADS):
                a = p[h]
                for j in (2 * h, 2 * h + 1):
                    acc[j] = acc[j] + a * rows_v[row, pl.ds(16 * j, 16)]
        invz = 1.0 / z
        for j in range(8):
            out_v[pl.ds(i * FEAT + 16 * j, 16)] = jnp.maximum(
                acc[j] * invz[j // 2], 0.0)


def _sc_body(taug_hbm, idx_hbm, out_hbm,
             spm, idx0, idx1, rows0, rows1, out0, out1,
             g0, g1, o0, o1, i0, i1):
    sid = lax.axis_index("s")
    wid = sid * NC + lax.axis_index("c")
    base = wid * NCHUNK
    lane = lax.iota(jnp.int32, 16)

    @pl.when(sid == 0)
    def _():
        pltpu.sync_copy(taug_hbm, spm)
    plsc.subcore_barrier()

    def idx_copy(ci, idx_v, sem):
        pltpu.async_copy(
            idx_hbm.at[pl.ds((base + ci) * IDXS, IDXS)], idx_v, sem)

    def idx_wait(idx_v, sem):
        pltpu.make_async_copy(idx_hbm.at[pl.ds(0, IDXS)], idx_v, sem).wait()

    def gather(idx_v, rows_v, sem):
        pltpu.async_copy(spm.at[idx_v], rows_v, sem)

    def gather_wait(rows_v, sem):
        pltpu.make_async_copy(
            taug_hbm.at[pl.ds(0, IDXS)], rows_v, sem).wait()

    idx_copy(0, idx0, i0)
    idx_copy(1, idx1, i1)
    idx_wait(idx0, i0)
    gather(idx0, rows0, g0)
    idx_wait(idx1, i1)
    gather(idx1, rows1, g1)

    npair = NCHUNK // 2
    bufs = ((idx0, rows0, out0, g0, o0, i0), (idx1, rows1, out1, g1, o1, i1))

    def pair(j, carry):
        for half, (idx_v, rows_v, out_v, gs, os, isem) in enumerate(bufs):
            ci = 2 * j + half
            gather_wait(rows_v, gs)

            @pl.when(j < npair - 1)
            def _():
                idx_copy(ci + 2, idx_v, isem)

            @pl.when(j > 0)
            def _():
                pltpu.make_async_copy(
                    out_v, out_hbm.at[pl.ds(0, GRP * FEAT)], os).wait()

            _compute_chunk(rows_v, out_v, lane)
            pltpu.async_copy(
                out_v,
                out_hbm.at[pl.ds((base + ci) * (GRP * FEAT), GRP * FEAT)], os)

            @pl.when(j < npair - 1)
            def _():
                idx_wait(idx_v, isem)
                gather(idx_v, rows_v, gs)
        return carry

    lax.fori_loop(0, npair, pair, 0)
    pltpu.make_async_copy(out0, out_hbm.at[pl.ds(0, GRP * FEAT)], o0).wait()
    pltpu.make_async_copy(out1, out_hbm.at[pl.ds(0, GRP * FEAT)], o1).wait()


@functools.partial(
    pl.kernel,
    out_type=jax.ShapeDtypeStruct((NPAD * FEAT,), jnp.float32),
    mesh=plsc.VectorSubcoreMesh(core_axis_name="c", subcore_axis_name="s"),
    scratch_types=[
        pltpu.VMEM_SHARED((N_NODES, DAUG), jnp.float32),
        pltpu.VMEM((IDXS,), jnp.int32),
        pltpu.VMEM((IDXS,), jnp.int32),
        pltpu.VMEM((IDXS, DAUG), jnp.float32),
        pltpu.VMEM((IDXS, DAUG), jnp.float32),
        pltpu.VMEM((GRP * FEAT,), jnp.float32),
        pltpu.VMEM((GRP * FEAT,), jnp.float32),
        pltpu.SemaphoreType.DMA,
        pltpu.SemaphoreType.DMA,
        pltpu.SemaphoreType.DMA,
        pltpu.SemaphoreType.DMA,
        pltpu.SemaphoreType.DMA,
        pltpu.SemaphoreType.DMA,
    ],
    compiler_params=pltpu.CompilerParams(use_tc_tiling_on_sc=False),
)
def _sc_attend(taug_hbm, idx_hbm, out_hbm, spm,
               idx0, idx1, rows0, rows1, out0, out1, g0, g1, o0, o1, i0, i1):
    _sc_body(taug_hbm, idx_hbm, out_hbm, spm,
             idx0, idx1, rows0, rows1, out0, out1, g0, g1, o0, o1, i0, i1)


def kernel(x, neighbors, Ws, As):
    n, f = x.shape
    h, o, _ = Ws.shape
    w_all = Ws.reshape(h * o, f)
    a_src = As[:, :o, 0]
    a_dst = As[:, o:, 0]
    afull = jnp.zeros((h * o, DAUG - FEAT), jnp.float32)
    for hh in range(h):
        afull = afull.at[hh * o:(hh + 1) * o, hh].set(a_dst[hh])
        afull = afull.at[hh * o:(hh + 1) * o, NHEADS + hh].set(a_src[hh])
    taug = _taug_matmul(x, w_all.T, afull)

    self_idx = jnp.arange(NPAD, dtype=jnp.int32)
    self_idx = jnp.where(self_idx < n, self_idx, 0)
    nbrs_pad = jnp.concatenate(
        [neighbors, jnp.zeros((NPAD - n, DEG), jnp.int32)], 0)
    idx33 = jnp.concatenate([self_idx[:, None], nbrs_pad], 1)
    idx_ch = idx33.reshape(NPAD // GRP, GRP * EDGES)
    idx_ch = jnp.pad(idx_ch, ((0, 0), (0, IDXS - GRP * EDGES)))
    idx_flat = idx_ch.reshape(-1)

    out_pad = _sc_attend(taug, idx_flat)
    return out_pad.reshape(NPAD, FEAT)[:n]


# speedup vs baseline: 1.5121x; 1.5121x over previous
import functools

import jax
import jax.numpy as jnp
from jax import lax
from jax.experimental import pallas as pl
from jax.experimental.pallas import tpu as pltpu
from jax.experimental.pallas import tpu_sc as plsc

N_NODES = 10000
DEG = 32
FEAT = 128
NHEADS = 4
OUT = 32
DAUG = 144
EDGES = DEG + 1

NC = 2
NS = 16
NW = NC * NS
GRP = 3
NT = 318
NPAD = NW * NT
NCHUNK = NT // GRP
IDXS = 104


def _mm_body(x_ref, w_ref, a_ref, o_ref):
    t = jnp.dot(x_ref[...], w_ref[...], preferred_element_type=jnp.float32)
    s = jnp.dot(t, a_ref[...], preferred_element_type=jnp.float32)
    o_ref[:, :FEAT] = t
    o_ref[:, FEAT:DAUG] = s


def _taug_matmul(x, w_all_t, afull):
    m, f = x.shape
    bm = 1000
    return pl.pallas_call(
        _mm_body,
        grid=(m // bm,),
        in_specs=[
            pl.BlockSpec((bm, f), lambda i: (i, 0)),
            pl.BlockSpec((f, FEAT), lambda i: (0, 0)),
            pl.BlockSpec((FEAT, DAUG - FEAT), lambda i: (0, 0)),
        ],
        out_specs=pl.BlockSpec((bm, DAUG), lambda i: (i, 0)),
        out_shape=jax.ShapeDtypeStruct((m, DAUG), jnp.float32),
    )(x, w_all_t, afull)


_LOG2E = 1.4426950408889634
_LN2_HI = 0.6931471824645996
_LN2_LO = -1.904654323148236e-09


def _exp_hi(v):
    t = v * _LOG2E
    tf = t + 0.5
    n = tf.astype(jnp.int32)
    nf = n.astype(jnp.float32)
    n = jnp.where(nf > tf, n - 1, n)
    nf = n.astype(jnp.float32)
    r = (v - nf * _LN2_HI) - nf * _LN2_LO
    p = jnp.float32(1.0 / 5040)
    for c in (1.0 / 720, 1.0 / 120, 1.0 / 24, 1.0 / 6, 0.5, 1.0, 1.0):
        p = p * r + c
    a = jnp.minimum(jnp.maximum(n + 30, 0), 30)
    b = jnp.minimum(jnp.maximum(n + 30 - a, 0), 30)
    one = jnp.full((16,), 1, jnp.int32)
    scale = (one << a).astype(jnp.float32) * (one << b).astype(jnp.float32)
    return (p * jnp.float32(2.0 ** -30)) * scale


def _periodic4(vec, tbuf, off):
    for s in (0, 4, 8, 12):
        tbuf[pl.ds(off + s, 16)] = vec
    return tbuf[pl.ds(off, 16)]


def _fold4(vec, tbuf, off, op):
    tbuf[pl.ds(off, 16)] = vec
    r = vec
    for s in (4, 8, 12):
        r = op(r, tbuf[pl.ds(off + s, 16)])
    return r


def _compute_chunk(rows_v, out_v, sbuf, tbuf, lane):
    mask4 = lane < NHEADS
    for i in range(GRP):
        r0 = i * EDGES
        srow_self = rows_v[r0, pl.ds(128, 16)]
        tbuf[pl.ds(0, 16)] = srow_self
        ssrc4 = tbuf[pl.ds(4, 16)]
        ssrc_p = _periodic4(ssrc4, tbuf, 16)
        sdst_p = _periodic4(srow_self, tbuf, 32)
        for k in range(EDGES):
            srow = rows_v[r0 + k, pl.ds(128, 16)]
            sbuf[pl.ds(4 * k, 16)] = srow
        sbuf[pl.ds(132, 16)] = sdst_p
        evs = []
        m = None
        for g in range(9):
            sc = ssrc_p + sbuf[pl.ds(16 * g, 16)]
            v = jnp.maximum(sc, 0.2 * sc)
            e = _exp_hi(v)
            evs.append(e)
            m = e if m is None else jnp.maximum(m, e)
        mf = _fold4(m, tbuf, 0, jnp.maximum)
        m_p = _periodic4(mf, tbuf, 16)
        ps = [jnp.exp(evs[g] - m_p) for g in range(9)]
        zv = jnp.where(mask4, ps[8], 0.0)
        for g in range(8):
            zv = zv + ps[g]
        zf = _fold4(zv, tbuf, 32, jnp.add)
        invz = 1.0 / zf
        acc = [jnp.zeros((16,), jnp.float32) for _ in range(8)]
        for k in range(EDGES):
            row = r0 + k
            for h in range(NHEADS):
                pos = 4 * k + h
                a = ps[pos >> 4][pos & 15]
                for j in (2 * h, 2 * h + 1):
                    acc[j] = acc[j] + a * rows_v[row, pl.ds(16 * j, 16)]
        for j in range(8):
            out_v[pl.ds(i * FEAT + 16 * j, 16)] = jnp.maximum(
                acc[j] * invz[j // 2], 0.0)


def _sc_body(taug_hbm, idx_hbm, out_hbm,
             spm, idx0, idx1, rows0, rows1, out0, out1, sbuf, tbuf,
             g0, g1, o0, o1, i0, i1):
    sid = lax.axis_index("s")
    wid = sid * NC + lax.axis_index("c")
    base = wid * NCHUNK
    lane = lax.iota(jnp.int32, 16)

    @pl.when(sid == 0)
    def _():
        pltpu.sync_copy(taug_hbm, spm)
    plsc.subcore_barrier()

    def idx_copy(ci, idx_v, sem):
        pltpu.async_copy(
            idx_hbm.at[pl.ds((base + ci) * IDXS, IDXS)], idx_v, sem)

    def idx_wait(idx_v, sem):
        pltpu.make_async_copy(idx_hbm.at[pl.ds(0, IDXS)], idx_v, sem).wait()

    def gather(idx_v, rows_v, sem):
        pltpu.async_copy(spm.at[idx_v], rows_v, sem)

    def gather_wait(rows_v, sem):
        pltpu.make_async_copy(
            taug_hbm.at[pl.ds(0, IDXS)], rows_v, sem).wait()

    idx_copy(0, idx0, i0)
    idx_copy(1, idx1, i1)
    idx_wait(idx0, i0)
    gather(idx0, rows0, g0)
    idx_wait(idx1, i1)
    gather(idx1, rows1, g1)

    npair = NCHUNK // 2
    bufs = ((idx0, rows0, out0, g0, o0, i0), (idx1, rows1, out1, g1, o1, i1))

    def pair(j, carry):
        for half, (idx_v, rows_v, out_v, gs, os, isem) in enumerate(bufs):
            ci = 2 * j + half
            gather_wait(rows_v, gs)

            @pl.when(j < npair - 1)
            def _():
                idx_copy(ci + 2, idx_v, isem)

            @pl.when(j > 0)
            def _():
                pltpu.make_async_copy(
                    out_v, out_hbm.at[pl.ds(0, GRP * FEAT)], os).wait()

            _compute_chunk(rows_v, out_v, sbuf, tbuf, lane)
            pltpu.async_copy(
                out_v,
                out_hbm.at[pl.ds((base + ci) * (GRP * FEAT), GRP * FEAT)], os)

            @pl.when(j < npair - 1)
            def _():
                idx_wait(idx_v, isem)
                gather(idx_v, rows_v, gs)
        return carry

    lax.fori_loop(0, npair, pair, 0)
    pltpu.make_async_copy(out0, out_hbm.at[pl.ds(0, GRP * FEAT)], o0).wait()
    pltpu.make_async_copy(out1, out_hbm.at[pl.ds(0, GRP * FEAT)], o1).wait()


@functools.partial(
    pl.kernel,
    out_type=jax.ShapeDtypeStruct((NPAD * FEAT,), jnp.float32),
    mesh=plsc.VectorSubcoreMesh(core_axis_name="c", subcore_axis_name="s"),
    scratch_types=[
        pltpu.VMEM_SHARED((N_NODES, DAUG), jnp.float32),
        pltpu.VMEM((IDXS,), jnp.int32),
        pltpu.VMEM((IDXS,), jnp.int32),
        pltpu.VMEM((IDXS, DAUG), jnp.float32),
        pltpu.VMEM((IDXS, DAUG), jnp.float32),
        pltpu.VMEM((GRP * FEAT,), jnp.float32),
        pltpu.VMEM((GRP * FEAT,), jnp.float32),
        pltpu.VMEM((160,), jnp.float32),
        pltpu.VMEM((64,), jnp.float32),
        pltpu.SemaphoreType.DMA,
        pltpu.SemaphoreType.DMA,
        pltpu.SemaphoreType.DMA,
        pltpu.SemaphoreType.DMA,
        pltpu.SemaphoreType.DMA,
        pltpu.SemaphoreType.DMA,
    ],
    compiler_params=pltpu.CompilerParams(use_tc_tiling_on_sc=False),
)
def _sc_attend(taug_hbm, idx_hbm, out_hbm, spm,
               idx0, idx1, rows0, rows1, out0, out1, sbuf, tbuf,
               g0, g1, o0, o1, i0, i1):
    _sc_body(taug_hbm, idx_hbm, out_hbm, spm,
             idx0, idx1, rows0, rows1, out0, out1, sbuf, tbuf,
             g0, g1, o0, o1, i0, i1)


def kernel(x, neighbors, Ws, As):
    n, f = x.shape
    h, o, _ = Ws.shape
    w_all = Ws.reshape(h * o, f)
    a_src = As[:, :o, 0]
    a_dst = As[:, o:, 0]
    afull = jnp.zeros((h * o, DAUG - FEAT), jnp.float32)
    for hh in range(h):
        afull = afull.at[hh * o:(hh + 1) * o, hh].set(a_dst[hh])
        afull = afull.at[hh * o:(hh + 1) * o, NHEADS + hh].set(a_src[hh])
    taug = _taug_matmul(x, w_all.T, afull)

    self_idx = jnp.arange(NPAD, dtype=jnp.int32)
    self_idx = jnp.where(self_idx < n, self_idx, 0)
    nbrs_pad = jnp.concatenate(
        [neighbors, jnp.zeros((NPAD - n, DEG), jnp.int32)], 0)
    idx33 = jnp.concatenate([self_idx[:, None], nbrs_pad], 1)
    idx_ch = idx33.reshape(NPAD // GRP, GRP * EDGES)
    idx_ch = jnp.pad(idx_ch, ((0, 0), (0, IDXS - GRP * EDGES)))
    idx_flat = idx_ch.reshape(-1)

    out_pad = _sc_attend(taug, idx_flat)
    return out_pad.reshape(NPAD, FEAT)[:n]
